# Initial kernel scaffold; baseline (speedup 1.0000x reference)
import jax
import jax.numpy as jnp
from jax.experimental import pallas as pl


def kernel(feat, src, dst, etypes, norm, nids, emb_table, W_size, b_size, W1, loop1, bias1, W2, loop2, bias2):
    raise NotImplementedError("write your pallas kernel here")


# speedup vs baseline: 17.1302x; 17.1302x over previous
import functools

import jax
import jax.numpy as jnp
from jax import lax
from jax.experimental import pallas as pl
from jax.experimental.pallas import tpu as pltpu
from jax.experimental.pallas import tpu_sc as plsc

N = 10000
E = 320000
MAX_LEN = 20
E_DIM = 8
VOCAB = 16
H = 128
NREL = 16
NB = 16

NC = 2
NS = 16
NW = NC * NS
EPW = E // NW
CH = 80
NCHUNK = EPW // CH
NP = 10240
RPT = NP // NS
ZR = 128

NBLK = 10
BN = N // NBLK


def _prep_body(feat_ref, a2_ref, bsz_ref, bd_ref, loop_ref, bias_ref,
               src_ref, et_ref, t_ref, xl_ref, g_ref):
    f = feat_ref[...]
    x = jnp.broadcast_to(bsz_ref[...], (BN, H))
    for j in range(MAX_LEN):
        ohj = (f[:, j:j + 1] == lax.broadcasted_iota(jnp.int32, (BN, VOCAB), 1)
               ).astype(jnp.float32)
        x = x + jnp.dot(ohj, a2_ref[j], preferred_element_type=jnp.float32)
    t_ref[...] = jnp.dot(x, bd_ref[...], preferred_element_type=jnp.float32)
    xl_ref[...] = jnp.dot(x, loop_ref[...],
                          preferred_element_type=jnp.float32) + bias_ref[...]
    g_ref[...] = src_ref[...] * NREL + et_ref[...]


def _mid_body(p0_ref, p1_ref, xl_ref, bd_ref, loop_ref, bias_ref,
              t_ref, xl2_ref):
    h = jnp.maximum(p0_ref[0] + p1_ref[0] + xl_ref[...], 0.0)
    t_ref[...] = jnp.dot(h, bd_ref[...], preferred_element_type=jnp.float32)
    xl2_ref[...] = jnp.dot(h, loop_ref[...],
                           preferred_element_type=jnp.float32) + bias_ref[...]


def _final_body(p0_ref, p1_ref, xl_ref, out_ref):
    out_ref[...] = p0_ref[0] + p1_ref[0] + xl_ref[...]


def _full(shape):
    return pl.BlockSpec(shape, lambda i: (0,) * len(shape))


ERC = 1000
ER = E // ERC
ERB = ER // NBLK

_prep_call = pl.pallas_call(
    _prep_body,
    grid=(NBLK,),
    in_specs=[
        pl.BlockSpec((BN, MAX_LEN), lambda i: (i, 0)),
        _full((MAX_LEN, VOCAB, H)),
        _full((1, H)),
        _full((H, NREL * H)),
        _full((H, H)),
        _full((1, H)),
        pl.BlockSpec((ERB, ERC), lambda i: (i, 0)),
        pl.BlockSpec((ERB, ERC), lambda i: (i, 0)),
    ],
    out_specs=[
        pl.BlockSpec((BN, NREL * H), lambda i: (i, 0)),
        pl.BlockSpec((BN, H), lambda i: (i, 0)),
        pl.BlockSpec((ERB, ERC), lambda i: (i, 0)),
    ],
    out_shape=[
        jax.ShapeDtypeStruct((N, NREL * H), jnp.float32),
        jax.ShapeDtypeStruct((N, H), jnp.float32),
        jax.ShapeDtypeStruct((ER, ERC), jnp.int32),
    ],
)

_mid_call = pl.pallas_call(
    _mid_body,
    grid=(NBLK,),
    in_specs=[
        pl.BlockSpec((1, BN, H), lambda i: (0, i, 0)),
        pl.BlockSpec((1, BN, H), lambda i: (1, i, 0)),
        pl.BlockSpec((BN, H), lambda i: (i, 0)),
        _full((H, NREL * H)),
        _full((H, H)),
        _full((1, H)),
    ],
    out_specs=[
        pl.BlockSpec((BN, NREL * H), lambda i: (i, 0)),
        pl.BlockSpec((BN, H), lambda i: (i, 0)),
    ],
    out_shape=[
        jax.ShapeDtypeStruct((N, NREL * H), jnp.float32),
        jax.ShapeDtypeStruct((N, H), jnp.float32),
    ],
)

_final_call = pl.pallas_call(
    _final_body,
    grid=(NBLK,),
    in_specs=[
        pl.BlockSpec((1, BN, H), lambda i: (0, i, 0)),
        pl.BlockSpec((1, BN, H), lambda i: (1, i, 0)),
        pl.BlockSpec((BN, H), lambda i: (i, 0)),
    ],
    out_specs=pl.BlockSpec((BN, H), lambda i: (i, 0)),
    out_shape=jax.ShapeDtypeStruct((N, H), jnp.float32),
)


def _edge_body(t_hbm, g_hbm, dst_hbm, norm_hbm, out_hbm,
               gsub, dsub, normsub, rows, acc, sem):
    c = lax.axis_index("c")
    s = lax.axis_index("s")
    wid = c * NS + s
    ebase = wid * EPW
    rbase = s * RPT

    def _zrow(i, _):
        for k in range(H // 16):
            rows[i, pl.ds(k * 16, 16)] = jnp.zeros((16,), jnp.float32)
        return 0
    lax.fori_loop(0, CH, _zrow, 0)
    for q in range(RPT // CH):
        pltpu.sync_copy(rows, acc.at[pl.ds(rbase + q * CH, CH)])
    plsc.subcore_barrier()

    def _chunk(i, _):
        off = ebase + i * CH
        pltpu.sync_copy(g_hbm.at[pl.ds(off, CH)], gsub)
        pltpu.sync_copy(dst_hbm.at[pl.ds(off, CH)], dsub)
        pltpu.sync_copy(norm_hbm.at[pl.ds(off, CH)], normsub)
        pltpu.async_copy(t_hbm.at[gsub], rows, sem).wait()

        def _scale(e, _):
            nb = plsc.load_gather(normsub, [jnp.full((16,), e, jnp.int32)])
            for k in range(H // 16):
                sl2 = pl.ds(k * 16, 16)
                rows[e, sl2] = rows[e, sl2] * nb
            return 0
        lax.fori_loop(0, CH, _scale, 0)

        pltpu.sync_copy(rows, acc.at[dsub], add=True)
        return 0

    lax.fori_loop(0, NCHUNK, _chunk, 0)
    plsc.subcore_barrier()

    pltpu.sync_copy(acc.at[pl.ds(rbase, RPT)],
                    out_hbm.at[c, pl.ds(rbase, RPT)])


_edge_call = pl.kernel(
    _edge_body,
    out_type=jax.ShapeDtypeStruct((NC, NP, H), jnp.float32),
    mesh=plsc.VectorSubcoreMesh(core_axis_name="c", subcore_axis_name="s"),
    compiler_params=pltpu.CompilerParams(needs_layout_passes=False),
    scratch_types=[
        pltpu.VMEM((CH,), jnp.int32),
        pltpu.VMEM((CH,), jnp.int32),
        pltpu.VMEM((CH,), jnp.float32),
        pltpu.VMEM((CH, H), jnp.float32),
        pltpu.VMEM_SHARED((NP, H), jnp.float32),
        pltpu.SemaphoreType.DMA,
    ],
)


def _bdcat(W):
    M = jnp.einsum('rbio,bc->rbico', W, jnp.eye(NB, dtype=W.dtype))
    BD = M.reshape(NREL, H, H)
    return BD.transpose(1, 0, 2).reshape(H, NREL * H)


def kernel(feat, src, dst, etypes, norm, nids, emb_table, W_size, b_size,
           W1, loop1, bias1, W2, loop2, bias2):
    A2 = jnp.einsum('vd,hjd->jvh', emb_table,
                    W_size.reshape(H, MAX_LEN, E_DIM))
    bd1 = _bdcat(W1)
    bd2 = _bdcat(W2)
    feat = feat.astype(jnp.int32)
    src = src.astype(jnp.int32)
    dst = dst.astype(jnp.int32)
    etypes = etypes.astype(jnp.int32)
    normf = norm.reshape(E)

    t1, xl1, g2d = _prep_call(feat, A2, b_size.reshape(1, H), bd1, loop1,
                              bias1.reshape(1, H), src.reshape(ER, ERC),
                              etypes.reshape(ER, ERC))
    g = g2d.reshape(E)
    p1 = _edge_call(t1.reshape(N * NREL, H), g, dst, normf)
    t2, xl2 = _mid_call(p1, p1, xl1, bd2, loop2, bias2.reshape(1, H))
    p2 = _edge_call(t2.reshape(N * NREL, H), g, dst, normf)
    return _final_call(p2, p2, xl2)


# speedup vs baseline: 40.4559x; 2.3617x over previous
import jax
import jax.numpy as jnp
from jax import lax
from jax.experimental import pallas as pl
from jax.experimental.pallas import tpu as pltpu
from jax.experimental.pallas import tpu_sc as plsc

N = 10000
E = 320000
MAX_LEN = 20
E_DIM = 8
VOCAB = 16
H = 128
NREL = 16
NB = 16

NC = 2
NS = 16
NW = NC * NS
EPW = E // NW
CH = 100
NCHUNK = EPW // CH
NCHT = E // CH
UNR = 5
NP = 10240
RPT = NP // NS

NBLK = 10
BN = N // NBLK


def _prep_body(feat_ref, a2_ref, bsz_ref, bd_ref, loop_ref, bias_ref,
               src_ref, et_ref, t_ref, xl_ref, g_ref, oh_ref):
    f = feat_ref[...]
    for j in range(MAX_LEN):
        oh_ref[:, j * VOCAB:(j + 1) * VOCAB] = (
            f[:, j:j + 1] == lax.broadcasted_iota(jnp.int32, (BN, VOCAB), 1)
        ).astype(jnp.float32)
    x = jnp.dot(oh_ref[...], a2_ref[...],
                preferred_element_type=jnp.float32) + bsz_ref[...]
    for r in range(NREL):
        t_ref[r] = jnp.dot(x, bd_ref[r], preferred_element_type=jnp.float32)
    xl_ref[...] = jnp.dot(x, loop_ref[...],
                          preferred_element_type=jnp.float32) + bias_ref[...]
    g_ref[...] = et_ref[...] * N + src_ref[...]


def _mid_body(p0_ref, p1_ref, xl_ref, bd_ref, loop_ref, bias_ref,
              t_ref, xl2_ref):
    h = jnp.maximum(p0_ref[0] + p1_ref[0] + xl_ref[...], 0.0)
    for r in range(NREL):
        t_ref[r] = jnp.dot(h, bd_ref[r], preferred_element_type=jnp.float32)
    xl2_ref[...] = jnp.dot(h, loop_ref[...],
                           preferred_element_type=jnp.float32) + bias_ref[...]


def _final_body(p0_ref, p1_ref, xl_ref, out_ref):
    out_ref[...] = p0_ref[0] + p1_ref[0] + xl_ref[...]


def _full(shape):
    return pl.BlockSpec(shape, lambda i: (0,) * len(shape))


ERC = 1000
ER = E // ERC
ERB = ER // NBLK

_prep_call = pl.pallas_call(
    _prep_body,
    grid=(NBLK,),
    in_specs=[
        pl.BlockSpec((BN, MAX_LEN), lambda i: (i, 0)),
        _full((MAX_LEN * VOCAB, H)),
        _full((1, H)),
        _full((NREL, H, H)),
        _full((H, H)),
        _full((1, H)),
        pl.BlockSpec((ERB, ERC), lambda i: (i, 0)),
        pl.BlockSpec((ERB, ERC), lambda i: (i, 0)),
    ],
    out_specs=[
        pl.BlockSpec((NREL, BN, H), lambda i: (0, i, 0)),
        pl.BlockSpec((BN, H), lambda i: (i, 0)),
        pl.BlockSpec((ERB, ERC), lambda i: (i, 0)),
    ],
    out_shape=[
        jax.ShapeDtypeStruct((NREL, N, H), jnp.float32),
        jax.ShapeDtypeStruct((N, H), jnp.float32),
        jax.ShapeDtypeStruct((ER, ERC), jnp.int32),
    ],
    scratch_shapes=[pltpu.VMEM((BN, MAX_LEN * VOCAB), jnp.float32)],
)

_mid_call = pl.pallas_call(
    _mid_body,
    grid=(NBLK,),
    in_specs=[
        pl.BlockSpec((1, BN, H), lambda i: (0, i, 0)),
        pl.BlockSpec((1, BN, H), lambda i: (1, i, 0)),
        pl.BlockSpec((BN, H), lambda i: (i, 0)),
        _full((NREL, H, H)),
        _full((H, H)),
        _full((1, H)),
    ],
    out_specs=[
        pl.BlockSpec((NREL, BN, H), lambda i: (0, i, 0)),
        pl.BlockSpec((BN, H), lambda i: (i, 0)),
    ],
    out_shape=[
        jax.ShapeDtypeStruct((NREL, N, H), jnp.float32),
        jax.ShapeDtypeStruct((N, H), jnp.float32),
    ],
)

_final_call = pl.pallas_call(
    _final_body,
    grid=(NBLK,),
    in_specs=[
        pl.BlockSpec((1, BN, H), lambda i: (0, i, 0)),
        pl.BlockSpec((1, BN, H), lambda i: (1, i, 0)),
        pl.BlockSpec((BN, H), lambda i: (i, 0)),
    ],
    out_specs=pl.BlockSpec((BN, H), lambda i: (i, 0)),
    out_shape=jax.ShapeDtypeStruct((N, H), jnp.float32),
)


def _edge_body(t_hbm, pck_hbm, zeros_hbm, out_hbm,
               ib0, ib1, ib2, rows0, rows1, rows2, acc, sem0, sem1, sem2):
    c = lax.axis_index("c")
    s = lax.axis_index("s")
    wid = c * NS + s
    cbase = wid * NCHUNK
    rbase = s * RPT
    ibs = (ib0, ib1, ib2)
    rowss = (rows0, rows1, rows2)
    sems = (sem0, sem1, sem2)

    pltpu.sync_copy(zeros_hbm, acc.at[pl.ds(rbase, RPT)])
    plsc.subcore_barrier()

    def _scale(ib, rows):
        def _body(q, _):
            for u in range(UNR):
                e = q * UNR + u
                ni = plsc.load_gather(
                    ib, [jnp.full((16,), 2, jnp.int32),
                         jnp.full((16,), e, jnp.int32)])
                nb = plsc.bitcast(ni, jnp.float32)
                for k in range(H // 16):
                    sl2 = pl.ds(k * 16, 16)
                    rows[e, sl2] = rows[e, sl2] * nb
            return 0
        lax.fori_loop(0, CH // UNR, _body, 0)

    def _step(i, X, first=False, prefetch=True):
        Z = (X + 2) % 3
        pltpu.make_async_copy(t_hbm.at[ibs[X].at[0]], rowss[X],
                              sems[X]).wait()
        _scale(ibs[X], rowss[X])
        if not first:
            pltpu.make_async_copy(rowss[Z], acc.at[ibs[Z].at[1]],
                                  sems[Z]).wait()
        if prefetch:
            pltpu.sync_copy(pck_hbm.at[i + 2], ibs[Z])
            pltpu.async_copy(t_hbm.at[ibs[Z].at[0]], rowss[Z], sems[Z])
        pltpu.async_copy(rowss[X], acc.at[ibs[X].at[1]], sems[X], add=True)

    pltpu.sync_copy(pck_hbm.at[cbase], ib0)
    pltpu.async_copy(t_hbm.at[ib0.at[0]], rows0, sem0)
    pltpu.sync_copy(pck_hbm.at[cbase + 1], ib1)
    pltpu.async_copy(t_hbm.at[ib1.at[0]], rows1, sem1)

    _step(cbase + 0, 0, first=True)
    _step(cbase + 1, 1)
    _step(cbase + 2, 2)

    def _triple(j, _):
        i0 = cbase + 3 * j
        _step(i0, 0)
        _step(i0 + 1, 1)
        _step(i0 + 2, 2)
        return 0
    lax.fori_loop(1, NCHUNK // 3 - 1, _triple, 0)

    _step(cbase + NCHUNK - 4, 0)
    _step(cbase + NCHUNK - 3, 1)
    _step(cbase + NCHUNK - 2, 2, prefetch=False)
    _step(cbase + NCHUNK - 1, 0, prefetch=False)
    pltpu.make_async_copy(rows0, acc.at[ib0.at[1]], sem0).wait()

    plsc.subcore_barrier()
    pltpu.sync_copy(acc.at[pl.ds(rbase, RPT)],
                    out_hbm.at[c, pl.ds(rbase, RPT)])


_edge_call = pl.kernel(
    _edge_body,
    out_type=jax.ShapeDtypeStruct((NC, NP, H), jnp.float32),
    mesh=plsc.VectorSubcoreMesh(core_axis_name="c", subcore_axis_name="s"),
    compiler_params=pltpu.CompilerParams(needs_layout_passes=False),
    scratch_types=[
        pltpu.VMEM((3, CH), jnp.int32),
        pltpu.VMEM((3, CH), jnp.int32),
        pltpu.VMEM((3, CH), jnp.int32),
        pltpu.VMEM((CH, H), jnp.float32),
        pltpu.VMEM((CH, H), jnp.float32),
        pltpu.VMEM((CH, H), jnp.float32),
        pltpu.VMEM_SHARED((NP, H), jnp.float32),
        pltpu.SemaphoreType.DMA,
        pltpu.SemaphoreType.DMA,
        pltpu.SemaphoreType.DMA,
    ],
)


def _bdcat(W):
    M = jnp.einsum('rbio,bc->rbico', W, jnp.eye(NB, dtype=W.dtype))
    return M.reshape(NREL, H, H)


def kernel(feat, src, dst, etypes, norm, nids, emb_table, W_size, b_size,
           W1, loop1, bias1, W2, loop2, bias2):
    A2 = jnp.einsum('vd,hjd->jvh', emb_table,
                    W_size.reshape(H, MAX_LEN, E_DIM)
                    ).reshape(MAX_LEN * VOCAB, H)
    bd1 = _bdcat(W1)
    bd2 = _bdcat(W2)
    feat = feat.astype(jnp.int32)
    src = src.astype(jnp.int32)
    dst = dst.astype(jnp.int32)
    etypes = etypes.astype(jnp.int32)
    normf = norm.reshape(E)

    t1, xl1, g2d = _prep_call(feat, A2, b_size.reshape(1, H), bd1, loop1,
                              bias1.reshape(1, H), src.reshape(ER, ERC),
                              etypes.reshape(ER, ERC))
    pck = jnp.stack([g2d.reshape(NCHT, CH), dst.reshape(NCHT, CH),
                     lax.bitcast_convert_type(normf, jnp.int32)
                     .reshape(NCHT, CH)], axis=1)
    zeros = jnp.zeros((RPT, H), jnp.float32)
    p1 = _edge_call(t1.reshape(NREL * N, H), pck, zeros)
    t2, xl2 = _mid_call(p1, p1, xl1, bd2, loop2, bias2.reshape(1, H))
    p2 = _edge_call(t2.reshape(NREL * N, H), pck, zeros)
    return _final_call(p2, p2, xl2)
